# baseline (speedup 1.0000x reference)
import jax
import jax.numpy as jnp
from jax.experimental import pallas as pl
from jax.experimental.pallas import tpu as pltpu

_BM = 512
_WCH = 2048


def _nearest_block(x_ref, c_ref, x2_ref, c2_ref, out_ref):
    x = x_ref[...]
    c = c_ref[...]
    dot = jax.lax.dot_general(
        x.astype(jnp.bfloat16), c.astype(jnp.bfloat16),
        (((1,), (1,)), ((), ())), preferred_element_type=jnp.float32
    )
    d2 = (x2_ref[...] + c2_ref[...]) - 2.0 * dot
    dist = jnp.sqrt(jnp.maximum(d2, 0.0))

    bm, k = dist.shape
    iota = jax.lax.broadcasted_iota(jnp.int32, (bm, _WCH), 1)
    acc = jnp.full((bm, 1), jnp.inf, jnp.float32)
    idx = jnp.zeros((bm, 1), jnp.int32)
    for j in range(k // _WCH):
        blk = dist[:, j * _WCH:(j + 1) * _WCH]
        m = jnp.min(blk, axis=1, keepdims=True)
        bi = jnp.min(jnp.where(blk == m, iota + j * _WCH, k),
                     axis=1, keepdims=True)
        steal = m < acc
        acc = jnp.where(steal, m.astype(jnp.bfloat16).astype(jnp.float32), acc)
        idx = jnp.where(steal, bi, idx)
    out_ref[0, 0, :] = idx[:, 0]


def kernel(features, cluster_centers):
    n, d = features.shape
    k, _ = cluster_centers.shape
    x2 = jnp.sum(features * features, axis=1, keepdims=True)
    c2 = jnp.sum(cluster_centers * cluster_centers, axis=1)[None, :]
    grid = n // _BM
    out = pl.pallas_call(
        _nearest_block,
        grid=(grid,),
        in_specs=[
            pl.BlockSpec((_BM, d), lambda i: (i, 0)),
            pl.BlockSpec((k, d), lambda i: (0, 0)),
            pl.BlockSpec((_BM, 1), lambda i: (i, 0)),
            pl.BlockSpec((1, k), lambda i: (0, 0)),
        ],
        out_specs=pl.BlockSpec((1, 1, _BM), lambda i: (i, 0, 0)),
        out_shape=jax.ShapeDtypeStruct((grid, 1, _BM), jnp.int32),
        compiler_params=pltpu.CompilerParams(
            dimension_semantics=("parallel",),
        ),
    )(features, cluster_centers, x2, c2)
    return out.reshape(n)


# speedup vs baseline: 1.5968x; 1.5968x over previous
import jax
import jax.numpy as jnp
from jax.experimental import pallas as pl
from jax.experimental.pallas import tpu as pltpu

_BM = 512
_WCH = 2048


def _nearest_block(x_ref, c_ref, x2_ref, c2_ref, out_ref):
    x = x_ref[...]
    c = c_ref[...]
    dot = jax.lax.dot_general(
        x.astype(jnp.bfloat16), c.astype(jnp.bfloat16),
        (((1,), (1,)), ((), ())), preferred_element_type=jnp.float32
    )
    d2 = (x2_ref[...] + c2_ref[...]) - 2.0 * dot

    bm, k = d2.shape
    iota = jax.lax.broadcasted_iota(jnp.int32, (bm, _WCH), 1)
    acc = jnp.full((bm, 1), jnp.inf, jnp.float32)
    idx = jnp.zeros((bm, 1), jnp.int32)
    for j in range(k // _WCH):
        blk = d2[:, j * _WCH:(j + 1) * _WCH]
        m2 = jnp.min(blk, axis=1, keepdims=True)
        bi = jnp.min(jnp.where(blk == m2, iota + j * _WCH, k),
                     axis=1, keepdims=True)
        m = jnp.sqrt(jnp.maximum(m2, 0.0))
        steal = m < acc
        acc = jnp.where(steal, m.astype(jnp.bfloat16).astype(jnp.float32), acc)
        idx = jnp.where(steal, bi, idx)
    out_ref[0, 0, :] = idx[:, 0]


def kernel(features, cluster_centers):
    n, d = features.shape
    k, _ = cluster_centers.shape
    x2 = jnp.sum(features * features, axis=1, keepdims=True)
    c2 = jnp.sum(cluster_centers * cluster_centers, axis=1)[None, :]
    grid = n // _BM
    out = pl.pallas_call(
        _nearest_block,
        grid=(grid,),
        in_specs=[
            pl.BlockSpec((_BM, d), lambda i: (i, 0)),
            pl.BlockSpec((k, d), lambda i: (0, 0)),
            pl.BlockSpec((_BM, 1), lambda i: (i, 0)),
            pl.BlockSpec((1, k), lambda i: (0, 0)),
        ],
        out_specs=pl.BlockSpec((1, 1, _BM), lambda i: (i, 0, 0)),
        out_shape=jax.ShapeDtypeStruct((grid, 1, _BM), jnp.int32),
        compiler_params=pltpu.CompilerParams(
            dimension_semantics=("parallel",),
        ),
    )(features, cluster_centers, x2, c2)
    return out.reshape(n)


# speedup vs baseline: 1.5973x; 1.0003x over previous
import jax
import jax.numpy as jnp
from jax.experimental import pallas as pl
from jax.experimental.pallas import tpu as pltpu

_BM = 512
_WCH = 2048


def _nearest_block(x_ref, c_ref, x2_ref, c2_ref, out_ref):
    x = x_ref[...]
    c = c_ref[...]
    dot = jax.lax.dot_general(
        x.astype(jnp.bfloat16), c.astype(jnp.bfloat16),
        (((1,), (1,)), ((), ())), preferred_element_type=jnp.float32
    )
    d2 = (x2_ref[...] + c2_ref[...]) - 2.0 * dot

    bm, k = d2.shape
    iota = jax.lax.broadcasted_iota(jnp.int32, (bm, _WCH), 1)
    acc = jnp.full((bm, 1), jnp.inf, jnp.float32)
    idx = jnp.zeros((bm, 1), jnp.int32)
    for j in range(k // _WCH):
        blk = d2[:, j * _WCH:(j + 1) * _WCH]
        m2 = jnp.min(blk, axis=1, keepdims=True)
        bi = jnp.min(jnp.where(blk == m2, iota, _WCH),
                     axis=1, keepdims=True) + j * _WCH
        m = jnp.sqrt(jnp.maximum(m2, 0.0))
        steal = m < acc
        acc = jnp.where(steal, m.astype(jnp.bfloat16).astype(jnp.float32), acc)
        idx = jnp.where(steal, bi, idx)
    out_ref[0, 0, :] = idx[:, 0]


def kernel(features, cluster_centers):
    n, d = features.shape
    k, _ = cluster_centers.shape
    x2 = jnp.sum(features * features, axis=1, keepdims=True)
    c2 = jnp.sum(cluster_centers * cluster_centers, axis=1)[None, :]
    grid = n // _BM
    out = pl.pallas_call(
        _nearest_block,
        grid=(grid,),
        in_specs=[
            pl.BlockSpec((_BM, d), lambda i: (i, 0)),
            pl.BlockSpec((k, d), lambda i: (0, 0)),
            pl.BlockSpec((_BM, 1), lambda i: (i, 0)),
            pl.BlockSpec((1, k), lambda i: (0, 0)),
        ],
        out_specs=pl.BlockSpec((1, 1, _BM), lambda i: (i, 0, 0)),
        out_shape=jax.ShapeDtypeStruct((grid, 1, _BM), jnp.int32),
        compiler_params=pltpu.CompilerParams(
            dimension_semantics=("parallel",),
        ),
    )(features, cluster_centers, x2, c2)
    return out.reshape(n)
